# baseline (speedup 1.0000x reference)
import functools

import jax
import jax.numpy as jnp
from jax import lax
from jax.experimental import pallas as pl
from jax.experimental.pallas import tpu as pltpu
from jax.experimental.pallas import tpu_sc as plsc

_N = 50000
_D = 256
_B = 1024
_K = 128
_L = 32

_NC = 2
_NS = 16
_NW = _NC * _NS
_GPW = _B // _NW
_LANES = 16
_CHUNKS = _D // _LANES


def _pooled_sparsecore(table, idx):
    mesh = plsc.VectorSubcoreMesh(core_axis_name="c", subcore_axis_name="s")

    @functools.partial(
        pl.kernel,
        mesh=mesh,
        out_type=jax.ShapeDtypeStruct((_B, _D), jnp.float32),
        scratch_types=[
            pltpu.VMEM((_GPW, _K), jnp.int32),
            pltpu.VMEM((6, _K // 2, _D), jnp.float32),
            pltpu.VMEM((_GPW, _D), jnp.float32),
            pltpu.SemaphoreType.DMA,
            pltpu.SemaphoreType.DMA,
            pltpu.SemaphoreType.DMA,
            pltpu.SemaphoreType.DMA,
            pltpu.SemaphoreType.DMA,
            pltpu.SemaphoreType.DMA,
        ],
    )
    def sc_kernel(table_hbm, idx_hbm, out_hbm, idx_v, rows_v, pooled_v,
                  *sems):
        wid = lax.axis_index("s") * _NC + lax.axis_index("c")
        base = wid * _GPW
        _U = _K // 2
        pltpu.sync_copy(idx_hbm.at[pl.ds(base, _GPW)], idx_v)

        def gather(g, h, slot):
            pltpu.async_copy(
                table_hbm.at[idx_v.at[g, pl.ds(h * _U, _U)]],
                rows_v.at[slot], sems[slot])

        def wait(g, h, slot):
            pltpu.make_async_copy(
                table_hbm.at[idx_v.at[g, pl.ds(h * _U, _U)]],
                rows_v.at[slot], sems[slot]).wait()

        def acc_half(slot, accs):
            buf = rows_v.at[slot]

            def body(r, accs):
                r2 = 2 * r
                return tuple(
                    accs[c]
                    + buf[r2, pl.ds(c * _LANES, _LANES)]
                    + buf[r2 + 1, pl.ds(c * _LANES, _LANES)]
                    for c in range(_CHUNKS)
                )

            return lax.fori_loop(0, _U // 2, body, accs)

        zeros = tuple(
            jnp.zeros((_LANES,), jnp.float32) for _ in range(_CHUNKS)
        )

        def do_graph(g, s0, s1, issue_ahead):
            wait(g, 0, s0)
            accs = acc_half(s0, zeros)
            if issue_ahead:
                gather(g + 3, 0, s0)
            wait(g, 1, s1)
            accs = acc_half(s1, accs)
            if issue_ahead:
                gather(g + 3, 1, s1)
            for c in range(_CHUNKS):
                pooled_v[g, pl.ds(c * _LANES, _LANES)] = accs[c]

        for g0 in range(3):
            gather(g0, 0, 2 * g0)
            gather(g0, 1, 2 * g0 + 1)

        def outer(t, _):
            g = 3 * t
            do_graph(g, 0, 1, True)
            do_graph(g + 1, 2, 3, True)
            do_graph(g + 2, 4, 5, True)
            return 0

        lax.fori_loop(0, _GPW // 3 - 1, outer, 0)
        do_graph(27, 0, 1, True)
        do_graph(28, 2, 3, True)
        do_graph(29, 4, 5, False)
        do_graph(30, 0, 1, False)
        do_graph(31, 2, 3, False)

        pltpu.sync_copy(pooled_v, out_hbm.at[pl.ds(base, _GPW)])

    return sc_kernel(table, idx)


def _head_kernel(pooled_ref, w_ref, b_ref, out_ref):
    logits = (
        jnp.dot(pooled_ref[...], w_ref[...],
                preferred_element_type=jnp.float32)
        + b_ref[...][None, :]
    )
    m = jnp.max(logits, axis=1, keepdims=True)
    shifted = logits - m
    lse = jnp.log(jnp.sum(jnp.exp(shifted), axis=1, keepdims=True))
    out_ref[...] = shifted - lse


def _head(pooled, W, b):
    return pl.pallas_call(
        _head_kernel,
        out_shape=jax.ShapeDtypeStruct((_B, _L), jnp.float32),
    )(pooled, W, b)


def kernel(node_embedding_matrix, batch_x_index, W, b):
    pooled = _pooled_sparsecore(node_embedding_matrix, batch_x_index)
    return _head(pooled, W, b)


# speedup vs baseline: 1.0233x; 1.0233x over previous
import functools

import jax
import jax.numpy as jnp
from jax import lax
from jax.experimental import pallas as pl
from jax.experimental.pallas import tpu as pltpu
from jax.experimental.pallas import tpu_sc as plsc

_N = 50000
_D = 256
_B = 1024
_K = 128
_L = 32

_NC = 2
_NS = 16
_NW = _NC * _NS
_GPW = _B // _NW
_LANES = 16
_CHUNKS = _D // _LANES


def _pooled_sparsecore(table, idx):
    mesh = plsc.VectorSubcoreMesh(core_axis_name="c", subcore_axis_name="s")

    @functools.partial(
        pl.kernel,
        mesh=mesh,
        out_type=jax.ShapeDtypeStruct((_B, _D), jnp.float32),
        scratch_types=[
            pltpu.VMEM((_GPW, _K), jnp.int32),
            pltpu.VMEM((3, _K, _D), jnp.float32),
            pltpu.VMEM((_GPW, _D), jnp.float32),
            pltpu.SemaphoreType.DMA,
            pltpu.SemaphoreType.DMA,
            pltpu.SemaphoreType.DMA,
        ],
    )
    def sc_kernel(table_hbm, idx_hbm, out_hbm, idx_v, rows_v, pooled_v,
                  sem0, sem1, sem2):
        sems = (sem0, sem1, sem2)
        wid = lax.axis_index("s") * _NC + lax.axis_index("c")
        base = wid * _GPW
        pltpu.sync_copy(idx_hbm.at[pl.ds(base, _GPW)], idx_v)

        def gather(j, slot):
            pltpu.async_copy(table_hbm.at[idx_v.at[j]], rows_v.at[slot],
                             sems[slot])

        def accumulate(j, slot):
            pltpu.make_async_copy(table_hbm.at[idx_v.at[j]],
                                  rows_v.at[slot], sems[slot]).wait()
            buf = rows_v.at[slot]

            def body(r, accs):
                r2 = 2 * r
                return tuple(
                    accs[c]
                    + buf[r2, pl.ds(c * _LANES, _LANES)]
                    + buf[r2 + 1, pl.ds(c * _LANES, _LANES)]
                    for c in range(_CHUNKS)
                )

            zeros = tuple(
                jnp.zeros((_LANES,), jnp.float32) for _ in range(_CHUNKS)
            )
            accs = lax.fori_loop(0, _K // 2, body, zeros)
            for c in range(_CHUNKS):
                pooled_v[j, pl.ds(c * _LANES, _LANES)] = accs[c]

        gather(0, 0)
        gather(1, 1)
        gather(2, 2)

        def outer(t, _):
            j0 = 3 * t
            for u in range(3):
                accumulate(j0 + u, u)
                gather(j0 + u + 3, u)
            return 0

        lax.fori_loop(0, _GPW // 3 - 1, outer, 0)
        accumulate(27, 0)
        gather(30, 0)
        accumulate(28, 1)
        gather(31, 1)
        accumulate(29, 2)
        accumulate(30, 0)
        accumulate(31, 1)

        pltpu.sync_copy(pooled_v, out_hbm.at[pl.ds(base, _GPW)])

    return sc_kernel(table, idx)


def _head_kernel(pooled_ref, w_ref, b_ref, out_ref):
    logits = (
        jnp.dot(pooled_ref[...], w_ref[...],
                preferred_element_type=jnp.float32)
        + b_ref[...][None, :]
    )
    m = jnp.max(logits, axis=1, keepdims=True)
    shifted = logits - m
    lse = jnp.log(jnp.sum(jnp.exp(shifted), axis=1, keepdims=True))
    out_ref[...] = shifted - lse


def _head(pooled, W, b):
    return pl.pallas_call(
        _head_kernel,
        out_shape=jax.ShapeDtypeStruct((_B, _L), jnp.float32),
    )(pooled, W, b)


def kernel(node_embedding_matrix, batch_x_index, W, b):
    pooled = _pooled_sparsecore(node_embedding_matrix, batch_x_index)
    return _head(pooled, W, b)
